# baseline (speedup 1.0000x reference)
import math

import jax
import jax.numpy as jnp
import numpy as np
from jax import lax
from jax.experimental import pallas as pl
from jax.experimental.pallas import tpu as pltpu
from jax.experimental.pallas import tpu_sc as plsc

N_NODES = 1000000
N_DIM = 16
R = 10.0
SIGMA = 1.0
BATCH = 16384

NW = 32
P = BATCH // NW
CH = 128
NCH = P // CH


def _log_C_D(n_dim, sigma, R_):
    r = np.linspace(1e-6, R_, 20001)
    log_integrand = (n_dim - 1) * (np.log1p(-np.exp(-2.0 * sigma * r)) + sigma * r - np.log(2.0))
    m = log_integrand.max()
    w = np.full(r.shape, r[1] - r[0])
    w[0] *= 0.5
    w[-1] *= 0.5
    return float(m + np.log(np.sum(w * np.exp(log_integrand - m))))


def _sum_log_I_D(n_dim):
    s = 0.0
    for j in range(1, n_dim - 1):
        m = n_dim - 1 - j
        I = math.sqrt(math.pi) * math.gamma((m + 1) / 2.0) / math.gamma(m / 2.0 + 1.0)
        s += math.log(I)
    return s


LOG_C_D = _log_C_D(N_DIM, SIGMA, R)
SUM_LOG_I_D = _sum_log_I_D(N_DIM)


def _sc_gather_stats(table, idx):
    mesh = plsc.VectorSubcoreMesh(
        core_axis_name="c", subcore_axis_name="s", num_cores=2, num_subcores=16
    )

    def body(table_hbm, idx_hbm, stats_hbm, iv, u0b, v0b, u1b, v1b, of, sem0, sem1):
        wid = lax.axis_index("s") * 2 + lax.axis_index("c")
        pltpu.sync_copy(idx_hbm.at[wid], iv)

        ubufs, vbufs, sems = (u0b, u1b), (v0b, v1b), (sem0, sem1)

        def fire(c, slot):
            du = pltpu.async_copy(
                table_hbm.at[iv.at[pl.ds(c * CH, CH)]], ubufs[slot], sems[slot])
            dv = pltpu.async_copy(
                table_hbm.at[iv.at[pl.ds(P + c * CH, CH)]], vbufs[slot], sems[slot])
            return du, dv

        ids = lax.iota(jnp.int32, 16)
        perms = [ids ^ sh for sh in (8, 4, 2, 1)]

        def bfly(x):
            for p in perms:
                x = x + x[p]
            return x

        def compute(c, uc, vc):
            def group(g, carry):
                z = jnp.zeros((16,), jnp.float32)
                lpa, nua, nva = z, z, z
                for j in range(16):
                    r = g * 16 + j
                    ua = uc[r, pl.ds(0, 16)]
                    va = vc[r, pl.ds(0, 16)]
                    ub = uc[r, pl.ds(1, 16)]
                    vb = vc[r, pl.ds(1, 16)]
                    pa0 = (ua * va)[0]
                    s1 = bfly(ub * vb)
                    s2 = bfly(ub * ub)
                    s3 = bfly(vb * vb)
                    sel = ids == j
                    lpa = jnp.where(sel, pa0 - s1, lpa)
                    nua = jnp.where(sel, s2, nua)
                    nva = jnp.where(sel, s3, nva)
                base = c * CH + g * 16
                of[pl.ds(base, 16)] = lpa
                of[pl.ds(P + base, 16)] = nua
                of[pl.ds(2 * P + base, 16)] = nva
                return carry

            lax.fori_loop(0, CH // 16, group, 0)

        descs = {0: fire(0, 0)}
        for c in range(NCH):
            if c + 1 < NCH:
                descs[c + 1] = fire(c + 1, (c + 1) % 2)
            du, dv = descs[c]
            du.wait()
            dv.wait()
            compute(c, ubufs[c % 2], vbufs[c % 2])

        pltpu.sync_copy(of, stats_hbm.at[wid])

    fn = pl.kernel(
        body,
        out_type=jax.ShapeDtypeStruct((NW, 3 * P), jnp.float32),
        mesh=mesh,
        compiler_params=pltpu.CompilerParams(use_tc_tiling_on_sc=False),
        scratch_types=[
            pltpu.VMEM((2 * P,), jnp.int32),
            pltpu.VMEM((CH, N_DIM + 1), jnp.float32),
            pltpu.VMEM((CH, N_DIM + 1), jnp.float32),
            pltpu.VMEM((CH, N_DIM + 1), jnp.float32),
            pltpu.VMEM((CH, N_DIM + 1), jnp.float32),
            pltpu.VMEM((3 * P,), jnp.float32),
            pltpu.SemaphoreType.DMA,
            pltpu.SemaphoreType.DMA,
        ],
    )
    return fn(table, idx)


def _arcosh(x):
    x = jnp.maximum(x, 1.0 + 1e-10)
    return jnp.log(x + jnp.sqrt(x * x - 1.0))


def _latent_lik_from_sq(sq):
    r = _arcosh(jnp.sqrt(1.0 + sq))
    r = jnp.where(r <= 1e-6, 1e-6, r)
    lik = -(N_DIM - 1) * (jnp.log(1.0 - jnp.exp(-2.0 * SIGMA * r) + 1e-5) + SIGMA * r - jnp.log(2.0))
    lik = lik + LOG_C_D
    lik = lik + SUM_LOG_I_D
    lik = lik + jnp.log(2.0 * jnp.pi)
    lik = lik + (N_DIM - 1) * (jnp.log(1.0 - jnp.exp(-2.0 * r) + 1e-5) + r - jnp.log(2.0))
    lik = lik + jnp.log(1.0 + jnp.exp(-2.0 * r) + 1e-5) + r - jnp.log(2.0)
    return lik


def _tc_body(lp_ref, nu_ref, nv_ref, lab_ref, bg_ref, out_ref):
    lp = lp_ref[...]
    beta = bg_ref[0, 0]
    gamma = bg_ref[0, 1]
    dist = _arcosh(lp)
    z = beta * dist - gamma
    loss = jnp.where(lab_ref[...] == 1,
                     jnp.logaddexp(0.0, z),
                     jnp.logaddexp(0.0, -z))
    lik = _latent_lik_from_sq(nu_ref[...]) + _latent_lik_from_sq(nv_ref[...])
    out_ref[...] = loss + lik / (N_NODES - 1)


def kernel(pairs, labels, table, beta, gamma):
    ui = pairs[:, 0].astype(jnp.int32).reshape(NW, P)
    vi = pairs[:, 1].astype(jnp.int32).reshape(NW, P)
    idx = jnp.concatenate([ui, vi], axis=1)

    table_lin = jnp.take(table, jnp.arange(N_NODES, dtype=jnp.int32), axis=0)
    stats = _sc_gather_stats(table_lin, idx)

    lp2d = stats[:, :P].reshape(128, 128)
    nu2d = stats[:, P:2 * P].reshape(128, 128)
    nv2d = stats[:, 2 * P:].reshape(128, 128)
    lab2d = labels.astype(jnp.int32).reshape(128, 128)
    bg = jnp.stack([beta.astype(jnp.float32), gamma.astype(jnp.float32)]).reshape(1, 2)

    out = pl.pallas_call(
        _tc_body,
        out_shape=jax.ShapeDtypeStruct((128, 128), jnp.float32),
        in_specs=[pl.BlockSpec(memory_space=pltpu.VMEM)] * 4
        + [pl.BlockSpec(memory_space=pltpu.SMEM)],
        out_specs=pl.BlockSpec(memory_space=pltpu.VMEM),
    )(lp2d, nu2d, nv2d, lab2d, bg)
    return out.reshape(BATCH)


# speedup vs baseline: 16.6821x; 16.6821x over previous
import math

import jax
import jax.numpy as jnp
import numpy as np
from jax import lax
from jax.experimental import pallas as pl
from jax.experimental.pallas import tpu as pltpu
from jax.experimental.pallas import tpu_sc as plsc

N_NODES = 1000000
N_DIM = 16
R = 10.0
SIGMA = 1.0
BATCH = 16384

NW = 32
P = BATCH // NW
CH = 128
NCH = P // CH


def _log_C_D(n_dim, sigma, R_):
    r = np.linspace(1e-6, R_, 20001)
    log_integrand = (n_dim - 1) * (np.log1p(-np.exp(-2.0 * sigma * r)) + sigma * r - np.log(2.0))
    m = log_integrand.max()
    w = np.full(r.shape, r[1] - r[0])
    w[0] *= 0.5
    w[-1] *= 0.5
    return float(m + np.log(np.sum(w * np.exp(log_integrand - m))))


def _sum_log_I_D(n_dim):
    s = 0.0
    for j in range(1, n_dim - 1):
        m = n_dim - 1 - j
        I = math.sqrt(math.pi) * math.gamma((m + 1) / 2.0) / math.gamma(m / 2.0 + 1.0)
        s += math.log(I)
    return s


LOG_C_D = _log_C_D(N_DIM, SIGMA, R)
SUM_LOG_I_D = _sum_log_I_D(N_DIM)


PADN = 1 << 20
DSH = 16
DBLK = 1 << DSH
NDB = PADN // DBLK


def _tc_detile_body(in_ref, out_ref):
    out_ref[...] = in_ref[...].reshape((N_DIM + 1) * DBLK)


def _tc_detile(table_tp):
    return pl.pallas_call(
        _tc_detile_body,
        grid=(NDB,),
        in_specs=[pl.BlockSpec((N_DIM + 1, DBLK), lambda b: (0, b))],
        out_specs=pl.BlockSpec(((N_DIM + 1) * DBLK,), lambda b: (b,)),
        out_shape=jax.ShapeDtypeStruct(((N_DIM + 1) * PADN,), jnp.float32),
    )(table_tp)


def _sc_gather_stats(table, idx):
    mesh = plsc.VectorSubcoreMesh(
        core_axis_name="c", subcore_axis_name="s", num_cores=2, num_subcores=16
    )

    ND = N_DIM + 1

    def body(table_hbm, idx_hbm, stats_hbm, iv, idxall, u0b, v0b, u1b, v1b, of,
             sem0, sem1):
        wid = lax.axis_index("s") * 2 + lax.axis_index("c")
        pltpu.sync_copy(idx_hbm.at[wid], iv)

        def precomp(sc, carry):
            for sub in range(CH // 16):
                o = sub * 16
                ivs = iv[pl.ds(sc * CH + o, 16)]
                b = lax.shift_right_logical(ivs, DSH)
                j = lax.bitwise_and(ivs, DBLK - 1)
                t0 = lax.shift_left(b * ND, DSH) + j
                for k in range(ND):
                    idxall[pl.ds(sc * (ND * CH) + k * CH + o, 16)] = t0 + (k << DSH)
            return carry

        lax.fori_loop(0, 2 * NCH, precomp, 0)

        ubufs, vbufs, sems = (u0b, u1b), (v0b, v1b), (sem0, sem1)

        def fire(c, slot):
            descs = []
            for k in range(ND):
                descs.append(pltpu.async_copy(
                    table_hbm.at[idxall.at[pl.ds(c * (ND * CH) + k * CH, CH)]],
                    ubufs[slot].at[pl.ds(k * CH, CH)], sems[slot]))
                descs.append(pltpu.async_copy(
                    table_hbm.at[idxall.at[pl.ds((NCH + c) * (ND * CH) + k * CH, CH)]],
                    vbufs[slot].at[pl.ds(k * CH, CH)], sems[slot]))
            return descs

        def compute(c, uc, vc):
            def sub(j, carry):
                o = j * 16
                u0 = uc[pl.ds(o, 16)]
                v0 = vc[pl.ds(o, 16)]
                lp = u0 * v0
                nu = jnp.zeros((16,), jnp.float32)
                nv = jnp.zeros((16,), jnp.float32)
                for k in range(1, ND):
                    ukv = uc[pl.ds(k * CH + o, 16)]
                    vkv = vc[pl.ds(k * CH + o, 16)]
                    lp = lp - ukv * vkv
                    nu = nu + ukv * ukv
                    nv = nv + vkv * vkv
                base = c * CH + j * 16
                of[pl.ds(base, 16)] = lp
                of[pl.ds(P + base, 16)] = nu
                of[pl.ds(2 * P + base, 16)] = nv
                return carry

            lax.fori_loop(0, CH // 16, sub, 0)

        descs = {0: fire(0, 0)}
        for c in range(NCH):
            if c + 1 < NCH:
                descs[c + 1] = fire(c + 1, (c + 1) % 2)
            for d in descs.pop(c):
                d.wait()
            compute(c, ubufs[c % 2], vbufs[c % 2])

        pltpu.sync_copy(of, stats_hbm.at[wid])

    fn = pl.kernel(
        body,
        out_type=jax.ShapeDtypeStruct((NW, 3 * P), jnp.float32),
        mesh=mesh,
        compiler_params=pltpu.CompilerParams(use_tc_tiling_on_sc=False),
        scratch_types=[
            pltpu.VMEM((2 * P,), jnp.int32),
            pltpu.VMEM((2 * NCH * ND * CH,), jnp.int32),
            pltpu.VMEM((ND * CH,), jnp.float32),
            pltpu.VMEM((ND * CH,), jnp.float32),
            pltpu.VMEM((ND * CH,), jnp.float32),
            pltpu.VMEM((ND * CH,), jnp.float32),
            pltpu.VMEM((3 * P,), jnp.float32),
            pltpu.SemaphoreType.DMA,
            pltpu.SemaphoreType.DMA,
        ],
    )
    return fn(table, idx)


def _arcosh(x):
    x = jnp.maximum(x, 1.0 + 1e-10)
    return jnp.log(x + jnp.sqrt(x * x - 1.0))


def _latent_lik_from_sq(sq):
    r = _arcosh(jnp.sqrt(1.0 + sq))
    r = jnp.where(r <= 1e-6, 1e-6, r)
    lik = -(N_DIM - 1) * (jnp.log(1.0 - jnp.exp(-2.0 * SIGMA * r) + 1e-5) + SIGMA * r - jnp.log(2.0))
    lik = lik + LOG_C_D
    lik = lik + SUM_LOG_I_D
    lik = lik + jnp.log(2.0 * jnp.pi)
    lik = lik + (N_DIM - 1) * (jnp.log(1.0 - jnp.exp(-2.0 * r) + 1e-5) + r - jnp.log(2.0))
    lik = lik + jnp.log(1.0 + jnp.exp(-2.0 * r) + 1e-5) + r - jnp.log(2.0)
    return lik


def _tc_body(lp_ref, nu_ref, nv_ref, lab_ref, bg_ref, out_ref):
    lp = lp_ref[...]
    beta = bg_ref[0, 0]
    gamma = bg_ref[0, 1]
    dist = _arcosh(lp)
    z = beta * dist - gamma
    loss = jnp.where(lab_ref[...] == 1,
                     jnp.logaddexp(0.0, z),
                     jnp.logaddexp(0.0, -z))
    lik = _latent_lik_from_sq(nu_ref[...]) + _latent_lik_from_sq(nv_ref[...])
    out_ref[...] = loss + lik / (N_NODES - 1)


def kernel(pairs, labels, table, beta, gamma):
    ui = pairs[:, 0].astype(jnp.int32).reshape(NW, P)
    vi = pairs[:, 1].astype(jnp.int32).reshape(NW, P)
    idx = jnp.concatenate([ui, vi], axis=1)

    tpad = jnp.pad(table.T, ((0, 0), (0, PADN - N_NODES)))
    flat = _tc_detile(tpad)
    stats = _sc_gather_stats(flat, idx)

    lp2d = stats[:, :P].reshape(128, 128)
    nu2d = stats[:, P:2 * P].reshape(128, 128)
    nv2d = stats[:, 2 * P:].reshape(128, 128)
    lab2d = labels.astype(jnp.int32).reshape(128, 128)
    bg = jnp.stack([beta.astype(jnp.float32), gamma.astype(jnp.float32)]).reshape(1, 2)

    out = pl.pallas_call(
        _tc_body,
        out_shape=jax.ShapeDtypeStruct((128, 128), jnp.float32),
        in_specs=[pl.BlockSpec(memory_space=pltpu.VMEM)] * 4
        + [pl.BlockSpec(memory_space=pltpu.SMEM)],
        out_specs=pl.BlockSpec(memory_space=pltpu.VMEM),
    )(lp2d, nu2d, nv2d, lab2d, bg)
    return out.reshape(BATCH)


# speedup vs baseline: 27.1593x; 1.6280x over previous
import math

import jax
import jax.numpy as jnp
import numpy as np
from jax import lax
from jax.experimental import pallas as pl
from jax.experimental.pallas import tpu as pltpu
from jax.experimental.pallas import tpu_sc as plsc

N_NODES = 1000000
N_DIM = 16
R = 10.0
SIGMA = 1.0
BATCH = 16384

NW = 32
P = BATCH // NW
CH = 128
NCH = P // CH


def _log_C_D(n_dim, sigma, R_):
    r = np.linspace(1e-6, R_, 20001)
    log_integrand = (n_dim - 1) * (np.log1p(-np.exp(-2.0 * sigma * r)) + sigma * r - np.log(2.0))
    m = log_integrand.max()
    w = np.full(r.shape, r[1] - r[0])
    w[0] *= 0.5
    w[-1] *= 0.5
    return float(m + np.log(np.sum(w * np.exp(log_integrand - m))))


def _sum_log_I_D(n_dim):
    s = 0.0
    for j in range(1, n_dim - 1):
        m = n_dim - 1 - j
        I = math.sqrt(math.pi) * math.gamma((m + 1) / 2.0) / math.gamma(m / 2.0 + 1.0)
        s += math.log(I)
    return s


LOG_C_D = _log_C_D(N_DIM, SIGMA, R)
SUM_LOG_I_D = _sum_log_I_D(N_DIM)


PADN = 1 << 20
DSH = 16
DBLK = 1 << DSH
NDB = PADN // DBLK


def _tc_detile_body(in_ref, out_ref):
    out_ref[...] = in_ref[...].reshape((N_DIM + 1) * DBLK)


def _tc_detile(table_tp):
    return pl.pallas_call(
        _tc_detile_body,
        grid=(NDB,),
        in_specs=[pl.BlockSpec((N_DIM + 1, DBLK), lambda b: (0, b))],
        out_specs=pl.BlockSpec(((N_DIM + 1) * DBLK,), lambda b: (b,)),
        out_shape=jax.ShapeDtypeStruct(((N_DIM + 1) * PADN,), jnp.float32),
    )(table_tp)


def _tc_detile_nopad(table_t):
    return pl.pallas_call(
        _tc_detile_body,
        grid=(NDB,),
        in_specs=[pl.BlockSpec((N_DIM + 1, DBLK), lambda b: (0, b))],
        out_specs=pl.BlockSpec(((N_DIM + 1) * DBLK,), lambda b: (b,)),
        out_shape=jax.ShapeDtypeStruct(((N_DIM + 1) * PADN,), jnp.float32),
    )(table_t)


def _sc_gather_stats(table, idx):
    mesh = plsc.VectorSubcoreMesh(
        core_axis_name="c", subcore_axis_name="s", num_cores=2, num_subcores=16
    )

    ND = N_DIM + 1

    def body(table_hbm, idx_hbm, stats_hbm, iv, idxall, u0b, v0b, u1b, v1b, of,
             sem0, sem1):
        wid = lax.axis_index("s") * 2 + lax.axis_index("c")
        pltpu.sync_copy(idx_hbm.at[wid], iv)

        def precomp(sc, carry):
            for sub in range(CH // 16):
                o = sub * 16
                ivs = iv[pl.ds(sc * CH + o, 16)]
                b = lax.shift_right_logical(ivs, DSH)
                j = lax.bitwise_and(ivs, DBLK - 1)
                t0 = lax.shift_left(b * ND, DSH) + j
                for k in range(ND):
                    idxall[pl.ds(sc * (ND * CH) + k * CH + o, 16)] = t0 + (k << DSH)
            return carry

        lax.fori_loop(0, 2 * NCH, precomp, 0)

        ubufs, vbufs, sems = (u0b, u1b), (v0b, v1b), (sem0, sem1)

        def fire(c, slot):
            descs = []
            for k in range(ND):
                descs.append(pltpu.async_copy(
                    table_hbm.at[idxall.at[pl.ds(c * (ND * CH) + k * CH, CH)]],
                    ubufs[slot].at[pl.ds(k * CH, CH)], sems[slot]))
                descs.append(pltpu.async_copy(
                    table_hbm.at[idxall.at[pl.ds((NCH + c) * (ND * CH) + k * CH, CH)]],
                    vbufs[slot].at[pl.ds(k * CH, CH)], sems[slot]))
            return descs

        def compute(c, uc, vc):
            def sub(j, carry):
                o = j * 16
                u0 = uc[pl.ds(o, 16)]
                v0 = vc[pl.ds(o, 16)]
                lp = u0 * v0
                nu = jnp.zeros((16,), jnp.float32)
                nv = jnp.zeros((16,), jnp.float32)
                for k in range(1, ND):
                    ukv = uc[pl.ds(k * CH + o, 16)]
                    vkv = vc[pl.ds(k * CH + o, 16)]
                    lp = lp - ukv * vkv
                    nu = nu + ukv * ukv
                    nv = nv + vkv * vkv
                base = c * CH + j * 16
                of[pl.ds(base, 16)] = lp
                of[pl.ds(P + base, 16)] = nu
                of[pl.ds(2 * P + base, 16)] = nv
                return carry

            lax.fori_loop(0, CH // 16, sub, 0)

        descs = {0: fire(0, 0)}
        for c in range(NCH):
            if c + 1 < NCH:
                descs[c + 1] = fire(c + 1, (c + 1) % 2)
            for d in descs.pop(c):
                d.wait()
            compute(c, ubufs[c % 2], vbufs[c % 2])

        pltpu.sync_copy(of, stats_hbm.at[wid])

    fn = pl.kernel(
        body,
        out_type=jax.ShapeDtypeStruct((NW, 3 * P), jnp.float32),
        mesh=mesh,
        compiler_params=pltpu.CompilerParams(use_tc_tiling_on_sc=False),
        scratch_types=[
            pltpu.VMEM((2 * P,), jnp.int32),
            pltpu.VMEM((2 * NCH * ND * CH,), jnp.int32),
            pltpu.VMEM((ND * CH,), jnp.float32),
            pltpu.VMEM((ND * CH,), jnp.float32),
            pltpu.VMEM((ND * CH,), jnp.float32),
            pltpu.VMEM((ND * CH,), jnp.float32),
            pltpu.VMEM((3 * P,), jnp.float32),
            pltpu.SemaphoreType.DMA,
            pltpu.SemaphoreType.DMA,
        ],
    )
    return fn(table, idx)


def _arcosh(x):
    x = jnp.maximum(x, 1.0 + 1e-10)
    return jnp.log(x + jnp.sqrt(x * x - 1.0))


def _latent_lik_from_sq(sq):
    r = _arcosh(jnp.sqrt(1.0 + sq))
    r = jnp.where(r <= 1e-6, 1e-6, r)
    lik = -(N_DIM - 1) * (jnp.log(1.0 - jnp.exp(-2.0 * SIGMA * r) + 1e-5) + SIGMA * r - jnp.log(2.0))
    lik = lik + LOG_C_D
    lik = lik + SUM_LOG_I_D
    lik = lik + jnp.log(2.0 * jnp.pi)
    lik = lik + (N_DIM - 1) * (jnp.log(1.0 - jnp.exp(-2.0 * r) + 1e-5) + r - jnp.log(2.0))
    lik = lik + jnp.log(1.0 + jnp.exp(-2.0 * r) + 1e-5) + r - jnp.log(2.0)
    return lik


def _tc_body(lp_ref, nu_ref, nv_ref, lab_ref, bg_ref, out_ref):
    lp = lp_ref[...]
    beta = bg_ref[0, 0]
    gamma = bg_ref[0, 1]
    dist = _arcosh(lp)
    z = beta * dist - gamma
    loss = jnp.where(lab_ref[...] == 1,
                     jnp.logaddexp(0.0, z),
                     jnp.logaddexp(0.0, -z))
    lik = _latent_lik_from_sq(nu_ref[...]) + _latent_lik_from_sq(nv_ref[...])
    out_ref[...] = loss + lik / (N_NODES - 1)


def kernel(pairs, labels, table, beta, gamma):
    ui = pairs[:, 0].astype(jnp.int32).reshape(NW, P)
    vi = pairs[:, 1].astype(jnp.int32).reshape(NW, P)
    idx = jnp.concatenate([ui, vi], axis=1)

    flat = _tc_detile_nopad(table.T)
    stats = _sc_gather_stats(flat, idx)

    lp2d = stats[:, :P].reshape(128, 128)
    nu2d = stats[:, P:2 * P].reshape(128, 128)
    nv2d = stats[:, 2 * P:].reshape(128, 128)
    lab2d = labels.astype(jnp.int32).reshape(128, 128)
    bg = jnp.stack([beta.astype(jnp.float32), gamma.astype(jnp.float32)]).reshape(1, 2)

    out = pl.pallas_call(
        _tc_body,
        out_shape=jax.ShapeDtypeStruct((128, 128), jnp.float32),
        in_specs=[pl.BlockSpec(memory_space=pltpu.VMEM)] * 4
        + [pl.BlockSpec(memory_space=pltpu.SMEM)],
        out_specs=pl.BlockSpec(memory_space=pltpu.VMEM),
    )(lp2d, nu2d, nv2d, lab2d, bg)
    return out.reshape(BATCH)
